# Initial kernel scaffold; baseline (speedup 1.0000x reference)
import jax
import jax.numpy as jnp
from jax.experimental import pallas as pl


def kernel(h_M, h_D, h_T, src_MvsD, dst_MvsD, src_DvsM, dst_DvsM, src_MvsT, dst_MvsT, src_TvsM, dst_TvsM, src_TvsD, dst_TvsD, src_DvsT, dst_DvsT, Adj, size, leftIndex, W1_MvsD, b1_MvsD, W1_DvsM, b1_DvsM, W1_MvsT, b1_MvsT, W1_TvsM, b1_TvsM, W1_TvsD, b1_TvsD, W1_DvsT, b1_DvsT, W2_MvsD, b2_MvsD, W2_DvsM, b2_DvsM, W2_MvsT, b2_MvsT, W2_TvsM, b2_TvsM, W2_TvsD, b2_TvsD, W2_DvsT, b2_DvsT, W3_MvsD, b3_MvsD, W3_DvsM, b3_DvsM, W3_MvsT, b3_MvsT, W3_TvsM, b3_TvsM, W3_TvsD, b3_TvsD, W3_DvsT, b3_DvsT, f1_w, f1_b, f2_w, f2_b, f3_w, f3_b, f4_w, f4_b):
    raise NotImplementedError("write your pallas kernel here")


# speedup vs baseline: 2.2484x; 2.2484x over previous
import functools

import jax
import jax.numpy as jnp
from jax import lax
from jax.experimental import pallas as pl
from jax.experimental.pallas import tpu as pltpu
from jax.experimental.pallas import tpu_sc as plsc

N_M, N_D, N_T = 50000, 5000, 45000
E = 100000
FEAT, HID, OUT = 128, 64, 64
ITEM = 5000
SIZE = 1024

NTILE = 16
NCORE = 2
EBLK = 128
NBLK = 49
TPB = NBLK * EBLK
EP = NTILE * TPB
PAD_DST = -(2 ** 30)

_mesh = plsc.VectorSubcoreMesh(core_axis_name="c", subcore_axis_name="s")
_sc_params = pltpu.CompilerParams(use_tc_tiling_on_sc=False)


DEG_DUMP = 212000
DEG_ACC = 212992
DEG_R0 = DEG_ACC
DEG_PER_TILE = 37504
DEG_NBLK = 293
DEG_LEN = NTILE * DEG_PER_TILE


def _deg_body(idx_hbm, out_hbm, acc, zbuf, ones_v, idxbuf):
    cid = lax.axis_index("c")
    sid = lax.axis_index("s")

    @pl.loop(0, 512)
    def _zinit(i):
        zbuf[pl.ds(i * 16, 16)] = jnp.zeros((16,), jnp.float32)

    for i in range(8):
        ones_v[pl.ds(i * 16, 16)] = jnp.ones((16,), jnp.float32)

    @pl.loop(sid, DEG_ACC // 8192, step=NTILE)
    def _zero(j):
        pltpu.sync_copy(zbuf, acc.at[pl.ds(j * 8192, 8192)])

    plsc.subcore_barrier()

    base = sid * DEG_PER_TILE

    @pl.loop(0, DEG_NBLK)
    def _scatter(b):
        off = base + b * EBLK
        pltpu.sync_copy(idx_hbm.at[cid, pl.ds(off, EBLK)], idxbuf)
        pltpu.sync_copy(ones_v, acc.at[idxbuf], add=True)

    plsc.subcore_barrier()

    @pl.loop(sid, DEG_ACC // 8192, step=NTILE)
    def _out(j):
        pltpu.sync_copy(acc.at[pl.ds(j * 8192, 8192)], zbuf)
        pltpu.sync_copy(zbuf, out_hbm.at[pl.ds(cid * DEG_ACC + j * 8192, 8192)])


_deg_call = functools.partial(
    pl.kernel,
    out_type=jax.ShapeDtypeStruct((2 * DEG_ACC,), jnp.float32),
    mesh=_mesh,
    compiler_params=_sc_params,
    scratch_types=[
        pltpu.VMEM_SHARED((DEG_ACC,), jnp.float32),
        pltpu.VMEM((8192,), jnp.float32),
        pltpu.VMEM((EBLK,), jnp.float32),
        pltpu.VMEM((EBLK,), jnp.int32),
    ],
)(_deg_body)


@functools.lru_cache(maxsize=None)
def _make_agg(rng):
    acc_rows = rng + 128
    dump = rng

    def body2(table, srcp, dstp, bases, out, acc, zbuf, basebuf, sidx, dbuf,
              dadj, rows, sem):
        cid = lax.axis_index("c")
        sid = lax.axis_index("s")

        @pl.loop(0, 128)
        def _zinit(i):
            for k in range(4):
                zbuf[i, pl.ds(k * 16, 16)] = jnp.zeros((16,), jnp.float32)

        @pl.loop(sid, acc_rows // 128, step=NTILE)
        def _zero(j):
            pltpu.sync_copy(zbuf, acc.at[pl.ds(j * 128, 128)])

        pltpu.sync_copy(bases.at[cid], basebuf)
        plsc.subcore_barrier()

        base16 = basebuf[...]
        ebase = sid * TPB

        @pl.loop(0, NBLK)
        def _block(b):
            off = ebase + b * EBLK
            pltpu.sync_copy(srcp.at[pl.ds(off, EBLK)], sidx)
            pltpu.sync_copy(dstp.at[pl.ds(off, EBLK)], dbuf)
            for k in range(8):
                d = dbuf[pl.ds(k * 16, 16)]
                local = d - base16
                valid = (local >= 0) & (local < rng)
                dadj[pl.ds(k * 16, 16)] = jnp.where(valid, local, dump)
            pltpu.async_copy(table.at[sidx], rows, sem).wait()
            pltpu.sync_copy(rows, acc.at[dadj], add=True)

        plsc.subcore_barrier()

        @pl.loop(sid, rng // 128, step=NTILE)
        def _out(j):
            pltpu.sync_copy(acc.at[pl.ds(j * 128, 128)], rows)
            pltpu.sync_copy(rows, out.at[pl.ds(cid * rng + j * 128, 128)])

    return pl.kernel(
        body2,
        out_type=jax.ShapeDtypeStruct((2 * rng, 64), jnp.float32),
        mesh=_mesh,
        compiler_params=_sc_params,
        scratch_types=[
            pltpu.VMEM_SHARED((acc_rows, 64), jnp.float32),
            pltpu.VMEM((128, 64), jnp.float32),
            pltpu.VMEM((16,), jnp.int32),
            pltpu.VMEM((EBLK,), jnp.int32),
            pltpu.VMEM((EBLK,), jnp.int32),
            pltpu.VMEM((EBLK,), jnp.int32),
            pltpu.VMEM((EBLK, 64), jnp.float32),
            pltpu.SemaphoreType.DMA,
        ],
    )


_PREC = lax.Precision.HIGHEST


def _mm2_body(x_ref, w_ref, da_ref, db_ref, ta_ref, tb_ref):
    y = jnp.dot(x_ref[...], w_ref[...], precision=_PREC,
                preferred_element_type=jnp.float32)
    sa = lax.rsqrt(jnp.maximum(da_ref[...], 1.0))
    sb = lax.rsqrt(jnp.maximum(db_ref[...], 1.0))
    ta_ref[...] = y[:, :64] * sa
    tb_ref[...] = y[:, 64:] * sb


@functools.lru_cache(maxsize=None)
def _make_mm2(n, f):
    br = 512
    grid = (n + br - 1) // br
    return pl.pallas_call(
        _mm2_body,
        grid=(grid,),
        in_specs=[
            pl.BlockSpec((br, f), lambda i: (i, 0)),
            pl.BlockSpec((f, 128), lambda i: (0, 0)),
            pl.BlockSpec((br, 1), lambda i: (i, 0)),
            pl.BlockSpec((br, 1), lambda i: (i, 0)),
        ],
        out_specs=[
            pl.BlockSpec((br, 64), lambda i: (i, 0)),
            pl.BlockSpec((br, 64), lambda i: (i, 0)),
        ],
        out_shape=[
            jax.ShapeDtypeStruct((n, 64), jnp.float32),
            jax.ShapeDtypeStruct((n, 64), jnp.float32),
        ],
    )


def _mm1_body(x_ref, w_ref, d_ref, t_ref):
    y = jnp.dot(x_ref[...], w_ref[...], precision=_PREC,
                preferred_element_type=jnp.float32)
    t_ref[...] = y * lax.rsqrt(jnp.maximum(d_ref[...], 1.0))


@functools.lru_cache(maxsize=None)
def _make_mm1(n, f):
    br = 512
    grid = (n + br - 1) // br
    return pl.pallas_call(
        _mm1_body,
        grid=(grid,),
        in_specs=[
            pl.BlockSpec((br, f), lambda i: (i, 0)),
            pl.BlockSpec((f, 64), lambda i: (0, 0)),
            pl.BlockSpec((br, 1), lambda i: (i, 0)),
        ],
        out_specs=pl.BlockSpec((br, 64), lambda i: (i, 0)),
        out_shape=jax.ShapeDtypeStruct((n, 64), jnp.float32),
    )


def _comb_body(aa_ref, ab_ref, da_ref, db_ref, ba_ref, bb_ref, o_ref):
    sa = lax.rsqrt(jnp.maximum(da_ref[...], 1.0))
    sb = lax.rsqrt(jnp.maximum(db_ref[...], 1.0))
    o_ref[...] = (aa_ref[...] * sa + ba_ref[...]
                  + ab_ref[...] * sb + bb_ref[...])


@functools.lru_cache(maxsize=None)
def _make_comb(n):
    br = 512
    grid = (n + br - 1) // br
    return pl.pallas_call(
        _comb_body,
        grid=(grid,),
        in_specs=[
            pl.BlockSpec((br, 64), lambda i: (i, 0)),
            pl.BlockSpec((br, 64), lambda i: (i, 0)),
            pl.BlockSpec((br, 1), lambda i: (i, 0)),
            pl.BlockSpec((br, 1), lambda i: (i, 0)),
            pl.BlockSpec((1, 64), lambda i: (0, 0)),
            pl.BlockSpec((1, 64), lambda i: (0, 0)),
        ],
        out_specs=pl.BlockSpec((br, 64), lambda i: (i, 0)),
        out_shape=jax.ShapeDtypeStruct((n, 64), jnp.float32),
    )


def _comb_norm_body(aa_ref, ab_ref, da_ref, db_ref, ba_ref, bb_ref, o_ref):
    sa = lax.rsqrt(jnp.maximum(da_ref[...], 1.0))
    sb = lax.rsqrt(jnp.maximum(db_ref[...], 1.0))
    y = (aa_ref[...] * sa + ba_ref[...] + ab_ref[...] * sb + bb_ref[...])
    nrm = jnp.maximum(jnp.sum(jnp.abs(y), axis=1, keepdims=True), 1e-12)
    o_ref[...] = y / nrm


_comb_norm = pl.pallas_call(
    _comb_norm_body,
    grid=(1,),
    in_specs=[
        pl.BlockSpec((SIZE, 64), lambda i: (0, 0)),
        pl.BlockSpec((SIZE, 64), lambda i: (0, 0)),
        pl.BlockSpec((SIZE, 1), lambda i: (0, 0)),
        pl.BlockSpec((SIZE, 1), lambda i: (0, 0)),
        pl.BlockSpec((1, 64), lambda i: (0, 0)),
        pl.BlockSpec((1, 64), lambda i: (0, 0)),
    ],
    out_specs=pl.BlockSpec((SIZE, 64), lambda i: (0, 0)),
    out_shape=jax.ShapeDtypeStruct((SIZE, 64), jnp.float32),
)


def _mlp_a_body(adj_ref, fk_ref, w1a_ref, w1b_ref, b1_ref, w2_ref, b2_ref,
                w3_ref, b3_ref, o_ref):
    x1 = jnp.dot(adj_ref[...], w1a_ref[...], precision=_PREC,
                 preferred_element_type=jnp.float32)
    x1 += jnp.dot(fk_ref[...], w1b_ref[...], precision=_PREC,
                  preferred_element_type=jnp.float32)
    x1 = jnp.maximum(x1 + b1_ref[...], 0.0)
    x2 = jnp.maximum(
        jnp.dot(x1, w2_ref[...], precision=_PREC,
                preferred_element_type=jnp.float32) + b2_ref[...], 0.0)
    o_ref[...] = jnp.maximum(
        jnp.dot(x2, w3_ref[...], precision=_PREC,
                preferred_element_type=jnp.float32) + b3_ref[...], 0.0)


_mlp_a = pl.pallas_call(
    _mlp_a_body,
    grid=(4,),
    in_specs=[
        pl.BlockSpec((256, ITEM), lambda i: (i, 0)),
        pl.BlockSpec((256, 64), lambda i: (i, 0)),
        pl.BlockSpec((ITEM, 256), lambda i: (0, 0)),
        pl.BlockSpec((64, 256), lambda i: (0, 0)),
        pl.BlockSpec((1, 256), lambda i: (0, 0)),
        pl.BlockSpec((256, 512), lambda i: (0, 0)),
        pl.BlockSpec((1, 512), lambda i: (0, 0)),
        pl.BlockSpec((512, 1024), lambda i: (0, 0)),
        pl.BlockSpec((1, 1024), lambda i: (0, 0)),
    ],
    out_specs=pl.BlockSpec((256, 1024), lambda i: (i, 0)),
    out_shape=jax.ShapeDtypeStruct((SIZE, 1024), jnp.float32),
)


def _mlp_b_body(x_ref, w_ref, b_ref, o_ref):
    y = jnp.dot(x_ref[...], w_ref[...], precision=_PREC,
                preferred_element_type=jnp.float32) + b_ref[...]
    o_ref[...] = jax.nn.sigmoid(y)


_mlp_b = pl.pallas_call(
    _mlp_b_body,
    grid=(4,),
    in_specs=[
        pl.BlockSpec((256, 1024), lambda i: (i, 0)),
        pl.BlockSpec((1024, ITEM), lambda i: (0, 0)),
        pl.BlockSpec((1, ITEM), lambda i: (0, 0)),
    ],
    out_specs=pl.BlockSpec((256, ITEM), lambda i: (i, 0)),
    out_shape=jax.ShapeDtypeStruct((SIZE, ITEM), jnp.float32),
)


def _pad_src(a):
    return jnp.concatenate([a, jnp.zeros((EP - E,), jnp.int32)])


def _pad_dst(a):
    return jnp.concatenate([a, jnp.full((EP - E,), PAD_DST, jnp.int32)])


def kernel(h_M, h_D, h_T, src_MvsD, dst_MvsD, src_DvsM, dst_DvsM, src_MvsT,
           dst_MvsT, src_TvsM, dst_TvsM, src_TvsD, dst_TvsD, src_DvsT,
           dst_DvsT, Adj, size, leftIndex, W1_MvsD, b1_MvsD, W1_DvsM, b1_DvsM,
           W1_MvsT, b1_MvsT, W1_TvsM, b1_TvsM, W1_TvsD, b1_TvsD, W1_DvsT,
           b1_DvsT, W2_MvsD, b2_MvsD, W2_DvsM, b2_DvsM, W2_MvsT, b2_MvsT,
           W2_TvsM, b2_TvsM, W2_TvsD, b2_TvsD, W2_DvsT, b2_DvsT, W3_MvsD,
           b3_MvsD, W3_DvsM, b3_DvsM, W3_MvsT, b3_MvsT, W3_TvsM, b3_TvsM,
           W3_TvsD, b3_TvsD, W3_DvsT, b3_DvsT, f1_w, f1_b, f2_w, f2_b, f3_w,
           f3_b, f4_w, f4_b):
    i32 = jnp.int32

    pad64 = jnp.full((64,), DEG_DUMP, i32)
    idx0 = jnp.concatenate([
        src_MvsD, dst_MvsD + 50000, src_DvsM + 55000, dst_DvsM + 60000,
        src_MvsT + 110000, dst_MvsT + 160000, pad64])
    idx1 = jnp.concatenate([
        src_TvsM, dst_TvsM + 45000, src_TvsD + 95000, dst_TvsD + 140000,
        src_DvsT + 145000, dst_DvsT + 150000, pad64])
    degs = _deg_call(jnp.stack([idx0, idx1]))

    def dslice(off, n):
        return lax.slice(degs, (off,), (off + n,)).reshape(n, 1)

    do_MvsD = dslice(0, N_M)
    di_MvsD = dslice(50000, N_D)
    do_DvsM = dslice(55000, N_D)
    di_DvsM = dslice(60000, N_M)
    do_MvsT = dslice(110000, N_M)
    di_MvsT = dslice(160000, N_T)
    do_TvsM = dslice(DEG_ACC + 0, N_T)
    di_TvsM = dslice(DEG_ACC + 45000, N_M)
    do_TvsD = dslice(DEG_ACC + 95000, N_T)
    di_TvsD = dslice(DEG_ACC + 140000, N_D)
    do_DvsT = dslice(DEG_ACC + 145000, N_D)
    di_DvsT = dslice(DEG_ACC + 150000, N_T)

    sp = {r: _pad_src(s) for r, s in [
        ("MvsD", src_MvsD), ("DvsM", src_DvsM), ("MvsT", src_MvsT),
        ("TvsM", src_TvsM), ("TvsD", src_TvsD), ("DvsT", src_DvsT)]}
    dp = {r: _pad_dst(d) for r, d in [
        ("MvsD", dst_MvsD), ("DvsM", dst_DvsM), ("MvsT", dst_MvsT),
        ("TvsM", dst_TvsM), ("TvsD", dst_TvsD), ("DvsT", dst_DvsT)]}

    def ceil128(x):
        return ((x + 127) // 128) * 128

    R_M, R_D, R_T, R_W = (ceil128(N_M // 2), ceil128(N_D // 2),
                          ceil128(N_T // 2), SIZE // 2)

    def bases_for(half):
        return jnp.tile(jnp.array([[0], [half]], i32), (1, 16))

    bases_M = bases_for(R_M)
    bases_D = bases_for(R_D)
    bases_T = bases_for(R_T)

    agg_M = _make_agg(R_M)
    agg_D = _make_agg(R_D)
    agg_T = _make_agg(R_T)
    agg_W = _make_agg(R_W)

    mm2_M = _make_mm2(N_M, FEAT)
    mm2_D = _make_mm2(N_D, FEAT)
    mm2_T = _make_mm2(N_T, FEAT)
    tM_D, tM_T = mm2_M(h_M, jnp.concatenate([W1_MvsD, W1_MvsT], 1),
                       do_MvsD, do_MvsT)
    tD_M, tD_T = mm2_D(h_D, jnp.concatenate([W1_DvsM, W1_DvsT], 1),
                       do_DvsM, do_DvsT)
    tT_M, tT_D = mm2_T(h_T, jnp.concatenate([W1_TvsM, W1_TvsD], 1),
                       do_TvsM, do_TvsD)

    a_MvsD = agg_D(tM_D, sp["MvsD"], dp["MvsD"], bases_D)
    a_DvsM = agg_M(tD_M, sp["DvsM"], dp["DvsM"], bases_M)
    a_MvsT = agg_T(tM_T, sp["MvsT"], dp["MvsT"], bases_T)
    a_TvsM = agg_M(tT_M, sp["TvsM"], dp["TvsM"], bases_M)
    a_TvsD = agg_D(tT_D, sp["TvsD"], dp["TvsD"], bases_D)
    a_DvsT = agg_T(tD_T, sp["DvsT"], dp["DvsT"], bases_T)

    comb_M = _make_comb(N_M)
    comb_D = _make_comb(N_D)
    comb_T = _make_comb(N_T)
    b = lambda v: v.reshape(1, 64)
    h1M = comb_M(a_DvsM, a_TvsM, di_DvsM, di_TvsM, b(b1_DvsM), b(b1_TvsM))
    h1D = comb_D(a_MvsD, a_TvsD, di_MvsD, di_TvsD, b(b1_MvsD), b(b1_TvsD))
    h1T = comb_T(a_MvsT, a_DvsT, di_MvsT, di_DvsT, b(b1_MvsT), b(b1_DvsT))

    mm2_Mh = _make_mm2(N_M, HID)
    mm1_T = _make_mm1(N_T, HID)
    mm1_D = _make_mm1(N_D, HID)
    t2M_D, t2M_T = mm2_Mh(h1M, jnp.concatenate([W2_MvsD, W2_MvsT], 1),
                          do_MvsD, do_MvsT)
    t2T_D = mm1_T(h1T, W2_TvsD, do_TvsD)
    t2D_T = mm1_D(h1D, W2_DvsT, do_DvsT)

    a2_MvsD = agg_D(t2M_D, sp["MvsD"], dp["MvsD"], bases_D)
    a2_TvsD = agg_D(t2T_D, sp["TvsD"], dp["TvsD"], bases_D)
    a2_MvsT = agg_T(t2M_T, sp["MvsT"], dp["MvsT"], bases_T)
    a2_DvsT = agg_T(t2D_T, sp["DvsT"], dp["DvsT"], bases_T)

    h2D = comb_D(a2_MvsD, a2_TvsD, di_MvsD, di_TvsD, b(b2_MvsD), b(b2_TvsD))
    h2T = comb_T(a2_MvsT, a2_DvsT, di_MvsT, di_DvsT, b(b2_MvsT), b(b2_DvsT))

    start = (jnp.asarray(leftIndex, i32)
             + jnp.asarray(size, i32) - jnp.int32(Adj.shape[0]))
    bases_w = jnp.tile(jnp.stack([start, start + SIZE // 2]).reshape(2, 1),
                       (1, 16))

    mm1_Dh = _make_mm1(N_D, HID)
    mm1_Th = _make_mm1(N_T, HID)
    t3D_M = mm1_Dh(h2D, W3_DvsM, do_DvsM)
    t3T_M = mm1_Th(h2T, W3_TvsM, do_TvsM)

    a3_D = agg_W(t3D_M, sp["DvsM"], dp["DvsM"], bases_w)
    a3_T = agg_W(t3T_M, sp["TvsM"], dp["TvsM"], bases_w)

    di_DvsM_w = lax.dynamic_slice(di_DvsM, (start, 0), (SIZE, 1))
    di_TvsM_w = lax.dynamic_slice(di_TvsM, (start, 0), (SIZE, 1))
    fake = _comb_norm(a3_D, a3_T, di_DvsM_w, di_TvsM_w,
                      b(b3_DvsM), b(b3_TvsM))

    x3 = _mlp_a(Adj, fake, f1_w[:ITEM], f1_w[ITEM:], f1_b.reshape(1, -1),
                f2_w, f2_b.reshape(1, -1), f3_w, f3_b.reshape(1, -1))
    x = _mlp_b(x3, f4_w, f4_b.reshape(1, -1))
    return (fake, x)
